# baseline (speedup 1.0000x reference)
import jax
import jax.numpy as jnp
from jax.experimental import pallas as pl
from jax.experimental.pallas import tpu as pltpu


def _make_body(nb, cs):
    def _fused_kernel(p0_ref, p1_ref, p2_ref,
                      f0, w10, b10, w20, b20,
                      f1, w11, b11, w21, b21,
                      f2, w12, b12, w22, b22,
                      o0, o1, o2,
                      m0_s, m1_s, m2_s):
        j = pl.program_id(1)

        @pl.when(j == 0)
        def _build_masks():
            for m_s, p_ref in ((m0_s, p0_ref), (m1_s, p1_ref), (m2_s, p2_ref)):
                npat, hw = m_s.shape
                pid_col = p_ref[...].reshape(npat, 1)
                iota = jax.lax.broadcasted_iota(jnp.int32, (npat, hw), 1)
                m_s[...] = (iota == pid_col).astype(jnp.bfloat16)

        for C, f, w1, b1, w2, b2, m_s, o in (
                (cs[0], f0, w10, b10, w20, b20, m0_s, o0),
                (cs[1], f1, w11, b11, w21, b21, m1_s, o1),
                (cs[2], f2, w12, b12, w22, b22, m2_s, o2)):
            w1b = w1[...].astype(jnp.bfloat16)
            w2b = w2[...].astype(jnp.bfloat16)
            x = f[0].astype(jnp.bfloat16)
            g = jnp.dot(m_s[...], x,
                        preferred_element_type=jnp.float32)
            gb = g.astype(jnp.bfloat16)
            gcat = jnp.concatenate([gb[:, b * C:(b + 1) * C]
                                    for b in range(nb)], axis=0)
            h = jnp.dot(gcat, w1b,
                        preferred_element_type=jnp.float32)
            h = jnp.maximum(h + b1[...], 0.0).astype(jnp.bfloat16)
            y = jnp.dot(h, w2b, preferred_element_type=jnp.float32)
            y = y + b2[...]
            norm = jnp.sqrt(jnp.sum(y * y, axis=-1, keepdims=True))
            yn = y / (norm + 1e-7)
            npat = yn.shape[0] // nb
            for b in range(nb):
                o[b] = yn[b * npat:(b + 1) * npat]

    return _fused_kernel


def kernel(feat0, pid0, w1_0, b1_0, w2_0, b2_0,
           feat1, pid1, w1_1, b1_1, w2_1, b2_1,
           feat2, pid2, w1_2, b1_2, w2_2, b2_2):
    B = feat0.shape[0]
    nc = w1_0.shape[1]
    P = pid0.shape[0]
    nb = 2 if B % 4 == 0 else 1
    steps = B // (2 * nb)

    pids, feats, wts, hws, cs = [], [], [], [], []
    flops = 0
    bytes_accessed = 0
    for feat, pid, w1, b1, w2, b2 in (
            (feat0, pid0, w1_0, b1_0, w2_0, b2_0),
            (feat1, pid1, w1_1, b1_1, w2_1, b2_1),
            (feat2, pid2, w1_2, b1_2, w2_2, b2_2)):
        C = feat.shape[1]
        hw = feat.shape[2] * feat.shape[3]
        cs.append(C)
        hws.append(hw)
        pids.append(pid.reshape(1, P))
        feats.append(jnp.transpose(feat.reshape(B // nb, nb, C, hw),
                                   (0, 3, 1, 2)).reshape(B // nb, hw, nb * C))
        wts.append((w1, b1.reshape(1, nc), w2, b2.reshape(1, nc)))
        flops += 2 * B * (P * hw * C + P * C * nc + P * nc * nc) + 5 * B * P * nc
        bytes_accessed += 4 * B * C * hw + 4 * B * P * nc + 4 * (C * nc + nc * nc)

    operands = list(pids)
    in_specs = [pl.BlockSpec((1, P), lambda c, j: (0, 0))] * 3
    for s in range(3):
        C, hw = cs[s], hws[s]
        w1, b1r, w2, b2r = wts[s]
        operands += [feats[s], w1, b1r, w2, b2r]
        in_specs += [
            pl.BlockSpec((1, hw, nb * C),
                         lambda c, j, st=steps: (c * st + j, 0, 0)),
            pl.BlockSpec((C, nc), lambda c, j: (0, 0)),
            pl.BlockSpec((1, nc), lambda c, j: (0, 0)),
            pl.BlockSpec((nc, nc), lambda c, j: (0, 0)),
            pl.BlockSpec((1, nc), lambda c, j: (0, 0)),
        ]

    outs = pl.pallas_call(
        _make_body(nb, cs),
        out_shape=[jax.ShapeDtypeStruct((B, P, nc), jnp.float32)] * 3,
        grid=(2, steps),
        in_specs=in_specs,
        out_specs=[pl.BlockSpec((nb, P, nc),
                                lambda c, j, st=steps: (c * st + j, 0, 0))] * 3,
        scratch_shapes=[pltpu.VMEM((P, hws[s]), jnp.bfloat16) for s in range(3)],
        compiler_params=pltpu.CompilerParams(
            dimension_semantics=("parallel", "arbitrary")),
        cost_estimate=pl.CostEstimate(
            flops=flops, transcendentals=B * P * 3,
            bytes_accessed=bytes_accessed),
    )(*operands)
    return list(outs)


# speedup vs baseline: 4.5776x; 4.5776x over previous
import jax
import jax.numpy as jnp
from jax.experimental import pallas as pl
from jax.experimental.pallas import tpu as pltpu

_BB = 2


def _make_body(bb):
    def _fused_kernel(p0_ref, p1_ref, p2_ref,
                      f0, w10, b10, w20, b20,
                      f1, w11, b11, w21, b21,
                      f2, w12, b12, w22, b22,
                      o0, o1, o2,
                      m0_s, m1_s, m2_s):
        j = pl.program_id(1)

        @pl.when(j == 0)
        def _build_masks():
            for m_s, p_ref in ((m0_s, p0_ref), (m1_s, p1_ref), (m2_s, p2_ref)):
                npat, hw = m_s.shape
                pid_col = p_ref[...].reshape(npat, 1)
                iota = jax.lax.broadcasted_iota(jnp.int32, (npat, hw), 1)
                m_s[...] = (iota == pid_col).astype(jnp.bfloat16)

        for f, w1, b1, w2, b2, m_s, o in (
                (f0, w10, b10, w20, b20, m0_s, o0),
                (f1, w11, b11, w21, b21, m1_s, o1),
                (f2, w12, b12, w22, b22, m2_s, o2)):
            w1b = w1[...].astype(jnp.bfloat16)
            w2b = w2[...].astype(jnp.bfloat16)
            C = f.shape[2]
            x = jnp.concatenate(
                [f[b].astype(jnp.bfloat16) for b in range(bb)], axis=1)
            g = jnp.dot(m_s[...], x,
                        preferred_element_type=jnp.float32)
            gb = g.astype(jnp.bfloat16)
            gcat = jnp.concatenate(
                [gb[:, b * C:(b + 1) * C] for b in range(bb)], axis=0)
            h = jnp.dot(gcat, w1b,
                        preferred_element_type=jnp.float32)
            h = jnp.maximum(h + b1[...], 0.0).astype(jnp.bfloat16)
            y = jnp.dot(h, w2b, preferred_element_type=jnp.float32)
            y = y + b2[...]
            norm = jnp.sqrt(jnp.sum(y * y, axis=-1, keepdims=True))
            yn = y / (norm + 1e-7)
            npat = yn.shape[0] // bb
            for b in range(bb):
                o[b] = yn[b * npat:(b + 1) * npat]

    return _fused_kernel


def kernel(feat0, pid0, w1_0, b1_0, w2_0, b2_0,
           feat1, pid1, w1_1, b1_1, w2_1, b2_1,
           feat2, pid2, w1_2, b1_2, w2_2, b2_2):
    B = feat0.shape[0]
    nc = w1_0.shape[1]
    P = pid0.shape[0]
    bb = max(1, min(_BB, B // 2))
    steps = B // (2 * bb)

    pids, feats, wts, hws, cs = [], [], [], [], []
    flops = 0
    bytes_accessed = 0
    for feat, pid, w1, b1, w2, b2 in (
            (feat0, pid0, w1_0, b1_0, w2_0, b2_0),
            (feat1, pid1, w1_1, b1_1, w2_1, b2_1),
            (feat2, pid2, w1_2, b1_2, w2_2, b2_2)):
        C = feat.shape[1]
        hw = feat.shape[2] * feat.shape[3]
        cs.append(C)
        hws.append(hw)
        pids.append(pid.reshape(1, P))
        feats.append(jnp.transpose(feat, (0, 2, 3, 1)).reshape(B, hw, C))
        wts.append((w1, b1.reshape(1, nc), w2, b2.reshape(1, nc)))
        flops += 2 * B * (P * hw * C + P * C * nc + P * nc * nc) + 5 * B * P * nc
        bytes_accessed += 2 * B * C * hw + 4 * B * P * nc + 4 * (C * nc + nc * nc)

    operands = list(pids)
    in_specs = [pl.BlockSpec((1, P), lambda c, j: (0, 0))] * 3
    for s in range(3):
        C, hw = cs[s], hws[s]
        w1, b1r, w2, b2r = wts[s]
        operands += [feats[s], w1, b1r, w2, b2r]
        in_specs += [
            pl.BlockSpec((bb, hw, C),
                         lambda c, j, st=steps: (c * st + j, 0, 0)),
            pl.BlockSpec((C, nc), lambda c, j: (0, 0)),
            pl.BlockSpec((1, nc), lambda c, j: (0, 0)),
            pl.BlockSpec((nc, nc), lambda c, j: (0, 0)),
            pl.BlockSpec((1, nc), lambda c, j: (0, 0)),
        ]

    outs = pl.pallas_call(
        _make_body(bb),
        out_shape=[jax.ShapeDtypeStruct((B, P, nc), jnp.float32)] * 3,
        grid=(2, steps),
        in_specs=in_specs,
        out_specs=[pl.BlockSpec((bb, P, nc),
                                lambda c, j, st=steps: (c * st + j, 0, 0))] * 3,
        scratch_shapes=[pltpu.VMEM((P, hws[s]), jnp.bfloat16) for s in range(3)],
        compiler_params=pltpu.CompilerParams(
            dimension_semantics=("parallel", "arbitrary")),
        cost_estimate=pl.CostEstimate(
            flops=flops, transcendentals=B * P * 3,
            bytes_accessed=bytes_accessed),
    )(*operands)
    return list(outs)


# speedup vs baseline: 4.5904x; 1.0028x over previous
import jax
import jax.numpy as jnp
from jax.experimental import pallas as pl
from jax.experimental.pallas import tpu as pltpu

_BB = 2


def _make_body(bb):
    def _fused_kernel(p0_ref, p1_ref, p2_ref,
                      f0, w10, b10, w20, b20,
                      f1, w11, b11, w21, b21,
                      f2, w12, b12, w22, b22,
                      o0, o1, o2,
                      m0_s, m1_s, m2_s):
        j = pl.program_id(1)

        @pl.when(j == 0)
        def _build_masks():
            for m_s, p_ref in ((m0_s, p0_ref), (m1_s, p1_ref), (m2_s, p2_ref)):
                npat, hw = m_s.shape
                pid_col = p_ref[...].reshape(npat, 1)
                iota = jax.lax.broadcasted_iota(jnp.int32, (npat, hw), 1)
                m_s[...] = (iota == pid_col).astype(jnp.bfloat16)

        for f, w1, b1, w2, b2, m_s, o in (
                (f0, w10, b10, w20, b20, m0_s, o0),
                (f1, w11, b11, w21, b21, m1_s, o1),
                (f2, w12, b12, w22, b22, m2_s, o2)):
            w1b = w1[...].astype(jnp.bfloat16)
            w2b = w2[...].astype(jnp.bfloat16)
            C = f.shape[2]
            x = jnp.concatenate(
                [f[b].astype(jnp.bfloat16) for b in range(bb)], axis=1)
            g = jnp.dot(m_s[...], x,
                        preferred_element_type=jnp.float32)
            gb = g.astype(jnp.bfloat16)
            gcat = jnp.concatenate(
                [gb[:, b * C:(b + 1) * C] for b in range(bb)], axis=0)
            h = jnp.dot(gcat, w1b,
                        preferred_element_type=jnp.float32)
            h = jnp.maximum(h + b1[...], 0.0).astype(jnp.bfloat16)
            y = jnp.dot(h, w2b, preferred_element_type=jnp.float32)
            y = y + b2[...]
            norm = jnp.sqrt(jnp.sum(y * y, axis=-1, keepdims=True))
            yn = y / (norm + 1e-7)
            npat = yn.shape[0] // bb
            for b in range(bb):
                o[b] = yn[b * npat:(b + 1) * npat]

    return _fused_kernel


def kernel(feat0, pid0, w1_0, b1_0, w2_0, b2_0,
           feat1, pid1, w1_1, b1_1, w2_1, b2_1,
           feat2, pid2, w1_2, b1_2, w2_2, b2_2):
    B = feat0.shape[0]
    nc = w1_0.shape[1]
    P = pid0.shape[0]
    bb = max(1, min(_BB, B // 2))
    steps = B // (2 * bb)

    pids, feats, wts, hws, cs = [], [], [], [], []
    flops = 0
    bytes_accessed = 0
    for feat, pid, w1, b1, w2, b2 in (
            (feat0, pid0, w1_0, b1_0, w2_0, b2_0),
            (feat1, pid1, w1_1, b1_1, w2_1, b2_1),
            (feat2, pid2, w1_2, b1_2, w2_2, b2_2)):
        C = feat.shape[1]
        hw = feat.shape[2] * feat.shape[3]
        cs.append(C)
        hws.append(hw)
        pids.append(pid.reshape(1, P))
        feats.append(jnp.transpose(feat, (0, 2, 3, 1)).reshape(B, hw, C))
        wts.append((w1, b1.reshape(1, nc), w2, b2.reshape(1, nc)))
        flops += 2 * B * (P * hw * C + P * C * nc + P * nc * nc) + 5 * B * P * nc
        bytes_accessed += 4 * B * C * hw + 4 * B * P * nc + 4 * (C * nc + nc * nc)

    operands = list(pids)
    in_specs = [pl.BlockSpec((1, P), lambda c, j: (0, 0))] * 3
    for s in range(3):
        C, hw = cs[s], hws[s]
        w1, b1r, w2, b2r = wts[s]
        operands += [feats[s], w1, b1r, w2, b2r]
        in_specs += [
            pl.BlockSpec((bb, hw, C),
                         lambda c, j, st=steps: (c * st + j, 0, 0)),
            pl.BlockSpec((C, nc), lambda c, j: (0, 0)),
            pl.BlockSpec((1, nc), lambda c, j: (0, 0)),
            pl.BlockSpec((nc, nc), lambda c, j: (0, 0)),
            pl.BlockSpec((1, nc), lambda c, j: (0, 0)),
        ]

    outs = pl.pallas_call(
        _make_body(bb),
        out_shape=[jax.ShapeDtypeStruct((B, P, nc), jnp.float32)] * 3,
        grid=(2, steps),
        in_specs=in_specs,
        out_specs=[pl.BlockSpec((bb, P, nc),
                                lambda c, j, st=steps: (c * st + j, 0, 0))] * 3,
        scratch_shapes=[pltpu.VMEM((P, hws[s]), jnp.bfloat16) for s in range(3)],
        compiler_params=pltpu.CompilerParams(
            dimension_semantics=("parallel", "arbitrary")),
        cost_estimate=pl.CostEstimate(
            flops=flops, transcendentals=B * P * 3,
            bytes_accessed=bytes_accessed),
    )(*operands)
    return list(outs)
